# baseline (speedup 1.0000x reference)
import functools

import jax
import jax.numpy as jnp
from jax import lax
from jax.experimental import pallas as pl
from jax.experimental.pallas import tpu as pltpu
from jax.experimental.pallas import tpu_sc as plsc

NNODES = 10000
NPAD = 10240
NC, NS = 2, 16
NW = NC * NS
CHUNK = 256
RPT = NPAD // NS

_MESH = plsc.VectorSubcoreMesh(
    core_axis_name="c", subcore_axis_name="s", num_cores=NC, num_subcores=NS
)


DW = 128


def _make_deg_kernel(epad, cpt):
    @functools.partial(
        pl.kernel,
        out_type=jax.ShapeDtypeStruct((NC * NPAD, DW), jnp.float32),
        mesh=_MESH,
        scratch_types=[
            pltpu.VMEM((CHUNK,), jnp.int32),
            pltpu.VMEM((CHUNK, DW), jnp.float32),
            pltpu.VMEM_SHARED((NPAD, DW), jnp.float32),
        ],
    )
    def deg_kernel(dst_hbm, ones_hbm, zeros_hbm, out_hbm, didx, ones_v, acc):
        c = lax.axis_index("c")
        s = lax.axis_index("s")
        rbase = s * RPT
        pltpu.sync_copy(zeros_hbm.at[pl.ds(rbase, RPT)], acc.at[pl.ds(rbase, RPT)])
        pltpu.sync_copy(ones_hbm, ones_v)
        plsc.subcore_barrier()
        ebase = c * (epad // NC) + s * (cpt * CHUNK)

        def body(i, carry):
            off = ebase + i * CHUNK
            pltpu.sync_copy(dst_hbm.at[pl.ds(off, CHUNK)], didx)
            pltpu.sync_copy(ones_v, acc.at[didx], add=True)
            return carry

        lax.fori_loop(0, cpt, body, 0)
        plsc.subcore_barrier()
        pltpu.sync_copy(
            acc.at[pl.ds(rbase, RPT)], out_hbm.at[pl.ds(c * NPAD + rbase, RPT)]
        )

    return deg_kernel


def _make_agg_kernel(epad, cpt, h):
    @functools.partial(
        pl.kernel,
        out_type=jax.ShapeDtypeStruct((NC * NPAD, h), jnp.float32),
        mesh=_MESH,
        scratch_types=[
            pltpu.VMEM((CHUNK,), jnp.int32),
            pltpu.VMEM((CHUNK,), jnp.int32),
            pltpu.VMEM((CHUNK, h), jnp.float32),
            pltpu.VMEM_SHARED((NPAD, h), jnp.float32),
            pltpu.SemaphoreType.DMA,
        ],
    )
    def agg_kernel(src_hbm, dst_hbm, g_hbm, zeros_hbm, out_hbm,
                   sidx, didx, rows, acc, sem):
        c = lax.axis_index("c")
        s = lax.axis_index("s")
        rbase = s * RPT
        pltpu.sync_copy(zeros_hbm.at[pl.ds(rbase, RPT)], acc.at[pl.ds(rbase, RPT)])
        plsc.subcore_barrier()
        ebase = c * (epad // NC) + s * (cpt * CHUNK)

        def body(i, carry):
            off = ebase + i * CHUNK
            pltpu.sync_copy(src_hbm.at[pl.ds(off, CHUNK)], sidx)
            pltpu.sync_copy(dst_hbm.at[pl.ds(off, CHUNK)], didx)
            pltpu.async_copy(g_hbm.at[sidx], rows, sem).wait()
            pltpu.sync_copy(rows, acc.at[didx], add=True)
            return carry

        lax.fori_loop(0, cpt, body, 0)
        plsc.subcore_barrier()
        pltpu.sync_copy(
            acc.at[pl.ds(rbase, RPT)], out_hbm.at[pl.ds(c * NPAD + rbase, RPT)]
        )

    return agg_kernel


def _tc1_body(cnt_ref, x_ref, w_ref, g_ref, dinv_ref):
    total = cnt_ref[:NPAD, 0:1] + cnt_ref[NPAD:, 0:1]
    dinv = lax.rsqrt(total + 1.0)
    dinv_ref[...] = dinv
    hmat = jnp.dot(x_ref[...], w_ref[...], preferred_element_type=jnp.float32)
    g_ref[...] = hmat * dinv


def _tc_mid_body(agg_ref, g_ref, dinv_ref, b_ref, w_ref, out_ref):
    dinv = dinv_ref[...]
    t = agg_ref[:NPAD] + agg_ref[NPAD:] + g_ref[...]
    hrel = jnp.maximum(t * dinv + b_ref[...], 0.0)
    out_ref[...] = (
        jnp.dot(hrel, w_ref[...], preferred_element_type=jnp.float32) * dinv
    )


def _tc_last_body(agg_ref, g_ref, dinv_ref, b_ref, out_ref):
    t = agg_ref[:NPAD] + agg_ref[NPAD:] + g_ref[...]
    out_ref[...] = jnp.maximum(t * dinv_ref[...] + b_ref[...], 0.0)


def kernel(x, edge_index, W1, b1, W2, b2):
    n, d = x.shape
    h = W1.shape[1]
    e = edge_index.shape[1]
    cpt = -(-e // (NW * CHUNK))
    epad = NW * cpt * CHUNK
    pad = epad - e

    src_p = jnp.concatenate([edge_index[0], jnp.full((pad,), n, jnp.int32)])
    dst_p = jnp.concatenate([edge_index[1], jnp.full((pad,), n, jnp.int32)])
    x_p = jnp.concatenate([x, jnp.zeros((NPAD - n, d), x.dtype)])
    zeros2 = jnp.zeros((NPAD, h), jnp.float32)
    zeros1 = zeros2 if DW == h else jnp.zeros((NPAD, DW), jnp.float32)
    ones1 = jnp.ones((CHUNK, DW), jnp.float32)

    deg_k = _make_deg_kernel(epad, cpt)
    agg_k = _make_agg_kernel(epad, cpt, h)

    cnt = deg_k(dst_p, ones1, zeros1)

    g1, dinv = pl.pallas_call(
        _tc1_body,
        out_shape=(
            jax.ShapeDtypeStruct((NPAD, h), jnp.float32),
            jax.ShapeDtypeStruct((NPAD, 1), jnp.float32),
        ),
    )(cnt, x_p, W1)

    agg1 = agg_k(src_p, dst_p, g1, zeros2)

    g2 = pl.pallas_call(
        _tc_mid_body,
        out_shape=jax.ShapeDtypeStruct((NPAD, h), jnp.float32),
    )(agg1, g1, dinv, b1.reshape(1, h), W2)

    agg2 = agg_k(src_p, dst_p, g2, zeros2)

    out = pl.pallas_call(
        _tc_last_body,
        out_shape=jax.ShapeDtypeStruct((NPAD, h), jnp.float32),
    )(agg2, g2, dinv, b2.reshape(1, h))

    return out[:n]


# speedup vs baseline: 1.1269x; 1.1269x over previous
import functools

import jax
import jax.numpy as jnp
from jax import lax
from jax.experimental import pallas as pl
from jax.experimental.pallas import tpu as pltpu
from jax.experimental.pallas import tpu_sc as plsc

NNODES = 10000
NPAD = 10240
NC, NS = 2, 16
NW = NC * NS
CHUNK = 64
RPT = NPAD // NS

_MESH = plsc.VectorSubcoreMesh(
    core_axis_name="c", subcore_axis_name="s", num_cores=NC, num_subcores=NS
)


DW = 128


def _make_deg_kernel(epad, cpt):
    @functools.partial(
        pl.kernel,
        out_type=jax.ShapeDtypeStruct((NC * NPAD, DW), jnp.float32),
        mesh=_MESH,
        scratch_types=[
            pltpu.VMEM((CHUNK,), jnp.int32),
            pltpu.VMEM((CHUNK, DW), jnp.float32),
            pltpu.VMEM_SHARED((NPAD, DW), jnp.float32),
        ],
    )
    def deg_kernel(dst_hbm, ones_hbm, zeros_hbm, out_hbm, didx, ones_v, acc):
        c = lax.axis_index("c")
        s = lax.axis_index("s")
        rbase = s * RPT
        pltpu.sync_copy(zeros_hbm.at[pl.ds(rbase, RPT)], acc.at[pl.ds(rbase, RPT)])
        pltpu.sync_copy(ones_hbm, ones_v)
        plsc.subcore_barrier()
        ebase = c * (epad // NC) + s * (cpt * CHUNK)

        def body(i, carry):
            off = ebase + i * CHUNK
            pltpu.sync_copy(dst_hbm.at[pl.ds(off, CHUNK)], didx)
            pltpu.sync_copy(ones_v, acc.at[didx], add=True)
            return carry

        lax.fori_loop(0, cpt, body, 0)
        plsc.subcore_barrier()
        pltpu.sync_copy(
            acc.at[pl.ds(rbase, RPT)], out_hbm.at[pl.ds(c * NPAD + rbase, RPT)]
        )

    return deg_kernel


def _make_agg_kernel(epad, cpt, h):
    @functools.partial(
        pl.kernel,
        out_type=jax.ShapeDtypeStruct((NC * NPAD, h), jnp.float32),
        mesh=_MESH,
        scratch_types=[
            pltpu.VMEM((CHUNK,), jnp.int32),
            pltpu.VMEM((CHUNK,), jnp.int32),
            pltpu.VMEM((CHUNK, h), jnp.float32),
            pltpu.VMEM_SHARED((NPAD, h), jnp.float32),
            pltpu.SemaphoreType.DMA,
        ],
    )
    def agg_kernel(src_hbm, dst_hbm, g_hbm, zeros_hbm, out_hbm,
                   sidx, didx, rows, acc, sem):
        c = lax.axis_index("c")
        s = lax.axis_index("s")
        rbase = s * RPT
        pltpu.sync_copy(zeros_hbm.at[pl.ds(rbase, RPT)], acc.at[pl.ds(rbase, RPT)])
        plsc.subcore_barrier()
        ebase = c * (epad // NC) + s * (cpt * CHUNK)

        def body(i, carry):
            off = ebase + i * CHUNK
            pltpu.sync_copy(src_hbm.at[pl.ds(off, CHUNK)], sidx)
            pltpu.sync_copy(dst_hbm.at[pl.ds(off, CHUNK)], didx)
            pltpu.async_copy(g_hbm.at[sidx], rows, sem).wait()
            pltpu.sync_copy(rows, acc.at[didx], add=True)
            return carry

        lax.fori_loop(0, cpt, body, 0)
        plsc.subcore_barrier()
        pltpu.sync_copy(
            acc.at[pl.ds(rbase, RPT)], out_hbm.at[pl.ds(c * NPAD + rbase, RPT)]
        )

    return agg_kernel


def _tc1_body(cnt_ref, x_ref, w_ref, g_ref, dinv_ref):
    total = cnt_ref[:NPAD, 0:1] + cnt_ref[NPAD:, 0:1]
    dinv = lax.rsqrt(total + 1.0)
    dinv_ref[...] = dinv
    hmat = jnp.dot(x_ref[...], w_ref[...], preferred_element_type=jnp.float32)
    g_ref[...] = hmat * dinv


def _tc_mid_body(agg_ref, g_ref, dinv_ref, b_ref, w_ref, out_ref):
    dinv = dinv_ref[...]
    t = agg_ref[:NPAD] + agg_ref[NPAD:] + g_ref[...]
    hrel = jnp.maximum(t * dinv + b_ref[...], 0.0)
    out_ref[...] = (
        jnp.dot(hrel, w_ref[...], preferred_element_type=jnp.float32) * dinv
    )


def _tc_last_body(agg_ref, g_ref, dinv_ref, b_ref, out_ref):
    t = agg_ref[:NPAD] + agg_ref[NPAD:] + g_ref[...]
    out_ref[...] = jnp.maximum(t * dinv_ref[...] + b_ref[...], 0.0)


def kernel(x, edge_index, W1, b1, W2, b2):
    n, d = x.shape
    h = W1.shape[1]
    e = edge_index.shape[1]
    cpt = -(-e // (NW * CHUNK))
    epad = NW * cpt * CHUNK
    pad = epad - e

    src_p = jnp.concatenate([edge_index[0], jnp.full((pad,), n, jnp.int32)])
    dst_p = jnp.concatenate([edge_index[1], jnp.full((pad,), n, jnp.int32)])
    x_p = jnp.concatenate([x, jnp.zeros((NPAD - n, d), x.dtype)])
    zeros2 = jnp.zeros((NPAD, h), jnp.float32)
    zeros1 = zeros2 if DW == h else jnp.zeros((NPAD, DW), jnp.float32)
    ones1 = jnp.ones((CHUNK, DW), jnp.float32)

    deg_k = _make_deg_kernel(epad, cpt)
    agg_k = _make_agg_kernel(epad, cpt, h)

    cnt = deg_k(dst_p, ones1, zeros1)

    g1, dinv = pl.pallas_call(
        _tc1_body,
        out_shape=(
            jax.ShapeDtypeStruct((NPAD, h), jnp.float32),
            jax.ShapeDtypeStruct((NPAD, 1), jnp.float32),
        ),
    )(cnt, x_p, W1)

    agg1 = agg_k(src_p, dst_p, g1, zeros2)

    g2 = pl.pallas_call(
        _tc_mid_body,
        out_shape=jax.ShapeDtypeStruct((NPAD, h), jnp.float32),
    )(agg1, g1, dinv, b1.reshape(1, h), W2)

    agg2 = agg_k(src_p, dst_p, g2, zeros2)

    out = pl.pallas_call(
        _tc_last_body,
        out_shape=jax.ShapeDtypeStruct((NPAD, h), jnp.float32),
    )(agg2, g2, dinv, b2.reshape(1, h))

    return out[:n]


# speedup vs baseline: 1.1599x; 1.0293x over previous
import functools

import jax
import jax.numpy as jnp
from jax import lax
from jax.experimental import pallas as pl
from jax.experimental.pallas import tpu as pltpu
from jax.experimental.pallas import tpu_sc as plsc

NNODES = 10000
NPAD = 10240
NC, NS = 2, 16
NW = NC * NS
CHUNK = 128
RPT = NPAD // NS

_MESH = plsc.VectorSubcoreMesh(
    core_axis_name="c", subcore_axis_name="s", num_cores=NC, num_subcores=NS
)


DW = 128


def _make_deg_kernel(epad, cpt):
    @functools.partial(
        pl.kernel,
        out_type=jax.ShapeDtypeStruct((NC * NPAD, DW), jnp.float32),
        mesh=_MESH,
        scratch_types=[
            pltpu.VMEM((CHUNK,), jnp.int32),
            pltpu.VMEM((CHUNK, DW), jnp.float32),
            pltpu.VMEM_SHARED((NPAD, DW), jnp.float32),
        ],
    )
    def deg_kernel(dst_hbm, ones_hbm, zeros_hbm, out_hbm, didx, ones_v, acc):
        c = lax.axis_index("c")
        s = lax.axis_index("s")
        rbase = s * RPT
        pltpu.sync_copy(zeros_hbm.at[pl.ds(rbase, RPT)], acc.at[pl.ds(rbase, RPT)])
        pltpu.sync_copy(ones_hbm, ones_v)
        plsc.subcore_barrier()
        ebase = c * (epad // NC) + s * (cpt * CHUNK)

        def body(i, carry):
            off = ebase + i * CHUNK
            pltpu.sync_copy(dst_hbm.at[pl.ds(off, CHUNK)], didx)
            pltpu.sync_copy(ones_v, acc.at[didx], add=True)
            return carry

        lax.fori_loop(0, cpt, body, 0)
        plsc.subcore_barrier()
        pltpu.sync_copy(
            acc.at[pl.ds(rbase, RPT)], out_hbm.at[pl.ds(c * NPAD + rbase, RPT)]
        )

    return deg_kernel


def _make_agg_kernel(epad, cpt, h):
    @functools.partial(
        pl.kernel,
        out_type=jax.ShapeDtypeStruct((NC * NPAD, h), jnp.float32),
        mesh=_MESH,
        scratch_types=[
            pltpu.VMEM((CHUNK,), jnp.int32),
            pltpu.VMEM((CHUNK,), jnp.int32),
            pltpu.VMEM((CHUNK, h), jnp.float32),
            pltpu.VMEM_SHARED((NPAD, h), jnp.float32),
            pltpu.SemaphoreType.DMA,
        ],
    )
    def agg_kernel(src_hbm, dst_hbm, g_hbm, zeros_hbm, out_hbm,
                   sidx, didx, rows, acc, sem):
        c = lax.axis_index("c")
        s = lax.axis_index("s")
        rbase = s * RPT
        pltpu.sync_copy(zeros_hbm.at[pl.ds(rbase, RPT)], acc.at[pl.ds(rbase, RPT)])
        plsc.subcore_barrier()
        ebase = c * (epad // NC) + s * (cpt * CHUNK)

        def body(i, carry):
            off = ebase + i * CHUNK
            pltpu.sync_copy(src_hbm.at[pl.ds(off, CHUNK)], sidx)
            pltpu.sync_copy(dst_hbm.at[pl.ds(off, CHUNK)], didx)
            pltpu.async_copy(g_hbm.at[sidx], rows, sem).wait()
            pltpu.sync_copy(rows, acc.at[didx], add=True)
            return carry

        lax.fori_loop(0, cpt, body, 0)
        plsc.subcore_barrier()
        pltpu.sync_copy(
            acc.at[pl.ds(rbase, RPT)], out_hbm.at[pl.ds(c * NPAD + rbase, RPT)]
        )

    return agg_kernel


def _tc1_body(cnt_ref, x_ref, w_ref, g_ref, dinv_ref):
    total = cnt_ref[:NPAD, 0:1] + cnt_ref[NPAD:, 0:1]
    dinv = lax.rsqrt(total + 1.0)
    dinv_ref[...] = dinv
    hmat = jnp.dot(x_ref[...], w_ref[...], preferred_element_type=jnp.float32)
    g_ref[...] = hmat * dinv


def _tc_mid_body(agg_ref, g_ref, dinv_ref, b_ref, w_ref, out_ref):
    dinv = dinv_ref[...]
    t = agg_ref[:NPAD] + agg_ref[NPAD:] + g_ref[...]
    hrel = jnp.maximum(t * dinv + b_ref[...], 0.0)
    out_ref[...] = (
        jnp.dot(hrel, w_ref[...], preferred_element_type=jnp.float32) * dinv
    )


def _tc_last_body(agg_ref, g_ref, dinv_ref, b_ref, out_ref):
    t = agg_ref[:NPAD] + agg_ref[NPAD:] + g_ref[...]
    out_ref[...] = jnp.maximum(t * dinv_ref[...] + b_ref[...], 0.0)


def kernel(x, edge_index, W1, b1, W2, b2):
    n, d = x.shape
    h = W1.shape[1]
    e = edge_index.shape[1]
    cpt = -(-e // (NW * CHUNK))
    epad = NW * cpt * CHUNK
    pad = epad - e

    src_p = jnp.concatenate([edge_index[0], jnp.full((pad,), n, jnp.int32)])
    dst_p = jnp.concatenate([edge_index[1], jnp.full((pad,), n, jnp.int32)])
    x_p = jnp.concatenate([x, jnp.zeros((NPAD - n, d), x.dtype)])
    zeros2 = jnp.zeros((NPAD, h), jnp.float32)
    zeros1 = zeros2 if DW == h else jnp.zeros((NPAD, DW), jnp.float32)
    ones1 = jnp.ones((CHUNK, DW), jnp.float32)

    deg_k = _make_deg_kernel(epad, cpt)
    agg_k = _make_agg_kernel(epad, cpt, h)

    cnt = deg_k(dst_p, ones1, zeros1)

    g1, dinv = pl.pallas_call(
        _tc1_body,
        out_shape=(
            jax.ShapeDtypeStruct((NPAD, h), jnp.float32),
            jax.ShapeDtypeStruct((NPAD, 1), jnp.float32),
        ),
    )(cnt, x_p, W1)

    agg1 = agg_k(src_p, dst_p, g1, zeros2)

    g2 = pl.pallas_call(
        _tc_mid_body,
        out_shape=jax.ShapeDtypeStruct((NPAD, h), jnp.float32),
    )(agg1, g1, dinv, b1.reshape(1, h), W2)

    agg2 = agg_k(src_p, dst_p, g2, zeros2)

    out = pl.pallas_call(
        _tc_last_body,
        out_shape=jax.ShapeDtypeStruct((NPAD, h), jnp.float32),
    )(agg2, g2, dinv, b2.reshape(1, h))

    return out[:n]
